# Initial kernel scaffold; baseline (speedup 1.0000x reference)
import jax
import jax.numpy as jnp
from jax.experimental import pallas as pl


def kernel(attention_scores, relative_position_bias_table, relative_position_index):
    raise NotImplementedError("write your pallas kernel here")


# speedup vs baseline: 2.6034x; 2.6034x over previous
import functools

import jax
import jax.numpy as jnp
from jax import lax
from jax.experimental import pallas as pl
from jax.experimental.pallas import tpu as pltpu
from jax.experimental.pallas import tpu_sc as plsc

SEQ = 577
NUM_HEADS = 12
NB_R = 8
R_BLK = 80
ROWS_PAD = NB_R * R_BLK
SP = 584
N_WORKERS = 32
N_TASKS = NUM_HEADS * NB_R
COL_OFFS = tuple(range(0, SP - 16, 16)) + (SP - 16,)


def _sc_gather_body(nrd, table_hbm, idx_hbm, pos_hbm, table_v, idx_v, out_v, sem):
    wid = lax.axis_index("s") * 2 + lax.axis_index("c")
    pltpu.sync_copy(table_hbm, table_v)
    for t in range(N_TASKS // N_WORKERS):
        task = wid + N_WORKERS * t
        h = task // NB_R
        rb = task % NB_R
        r0 = rb * R_BLK
        pltpu.sync_copy(idx_hbm.at[pl.ds(r0, R_BLK), :], idx_v)
        hoff = h * nrd

        def row_body(r, _, hoff=hoff):
            for off in COL_OFFS:
                idx16 = idx_v[r, pl.ds(off, 16)]
                g = plsc.load_gather(table_v, [idx16 + hoff])
                out_v[r, pl.ds(off, 16)] = g
            return 0

        lax.fori_loop(0, R_BLK, row_body, 0)
        pltpu.sync_copy(out_v, pos_hbm.at[h, pl.ds(r0, R_BLK), :])


def _sc_gather(table_t_flat, idx_pad, nrd):
    mesh = plsc.VectorSubcoreMesh(core_axis_name="c", subcore_axis_name="s")
    fn = functools.partial(
        pl.kernel,
        mesh=mesh,
        out_type=jax.ShapeDtypeStruct((NUM_HEADS, ROWS_PAD, SP), jnp.float32),
        scratch_types=[
            pltpu.VMEM((NUM_HEADS * nrd,), jnp.float32),
            pltpu.VMEM((R_BLK, SP), jnp.int32),
            pltpu.VMEM((R_BLK, SP), jnp.float32),
            pltpu.SemaphoreType.DMA,
        ],
        compiler_params=pltpu.CompilerParams(needs_layout_passes=False),
    )(functools.partial(_sc_gather_body, nrd))
    return fn(table_t_flat, idx_pad)


def _add_body(a_ref, p_ref, o_ref):
    o_ref[...] = a_ref[...] + p_ref[:, :SEQ, :SEQ][None]


def _tc_add(attn, pos_pad):
    b, nh, s, _ = attn.shape
    return pl.pallas_call(
        _add_body,
        grid=(nh,),
        in_specs=[
            pl.BlockSpec((b, 1, s, s), lambda h: (0, h, 0, 0)),
            pl.BlockSpec((1, SP, SP), lambda h: (h, 0, 0)),
        ],
        out_specs=pl.BlockSpec((b, 1, s, s), lambda h: (0, h, 0, 0)),
        out_shape=jax.ShapeDtypeStruct(attn.shape, attn.dtype),
        compiler_params=pltpu.CompilerParams(
            vmem_limit_bytes=110 * 1024 * 1024,
        ),
    )(attn, pos_pad)


def kernel(attention_scores, relative_position_bias_table, relative_position_index):
    nrd = relative_position_bias_table.shape[0]
    table_t_flat = jnp.transpose(relative_position_bias_table).reshape(-1)
    idx_pad = jnp.pad(
        relative_position_index,
        ((0, ROWS_PAD - SEQ), (0, SP - SEQ)),
    )
    pos_pad = _sc_gather(table_t_flat, idx_pad, nrd)
    return _tc_add(attention_scores, pos_pad)
